# baseline (speedup 1.0000x reference)
import jax
import jax.numpy as jnp
from jax import lax
from jax.experimental import pallas as pl
from jax.experimental.pallas import tpu as pltpu
from jax.experimental.pallas import tpu_sc as plsc

_NC = 2
_NS = 16
_NW = _NC * _NS
_L = 16
_CH = 128


def _sc_body(x_hbm, pidx_hbm, cidx_hbm, uidx_hbm, duv_hbm, duw_hbm,
             pidx_v, cidx_v, uidx_v, pu0, pv0, pw0, pu1, pv1, pw1,
             duv_v, duw_v, sem):
    D = x_hbm.shape[0]
    n_chunks = pidx_v.shape[0]
    bpw = n_chunks * _CH
    wid = lax.axis_index("s") * _NC + lax.axis_index("c")
    base_chunk = wid * n_chunks

    pltpu.sync_copy(pidx_hbm.at[pl.ds(base_chunk, n_chunks)], pidx_v)
    pltpu.sync_copy(cidx_hbm.at[pl.ds(base_chunk, n_chunks)], cidx_v)
    pltpu.sync_copy(uidx_hbm.at[pl.ds(base_chunk, n_chunks)], uidx_v)

    bufs = ((pu0, pv0, pw0), (pu1, pv1, pw1))

    def fire(j):
        pu, pv, pw = bufs[j % 2]
        descs = []
        for d in range(D):
            descs.append(pltpu.async_copy(
                x_hbm.at[d].at[pidx_v.at[j]], pu.at[d], sem))
            descs.append(pltpu.async_copy(
                x_hbm.at[d].at[cidx_v.at[j]], pv.at[d], sem))
            descs.append(pltpu.async_copy(
                x_hbm.at[d].at[uidx_v.at[j]], pw.at[d], sem))
        return descs

    n_sl = _CH // _L
    cur = fire(0)
    for j in range(n_chunks):
        for dsc in cur:
            dsc.wait()
        if j + 1 < n_chunks:
            cur = fire(j + 1)
        pu, pv, pw = bufs[j % 2]

        t_uv = []
        t_uw = []
        for s in range(n_sl):
            sl = pl.ds(s * _L, _L)
            u0 = pu.at[0][sl]
            t_uv.append(u0 * pv.at[0][sl])
            t_uw.append(u0 * pw.at[0][sl])

        def body(d, accs, pu=pu, pv=pv, pw=pw):
            new_uv = []
            new_uw = []
            for s in range(n_sl):
                sl = pl.ds(s * _L, _L)
                ud = pu[d, sl]
                new_uv.append(accs[s] + ud * pv[d, sl])
                new_uw.append(accs[n_sl + s] + ud * pw[d, sl])
            return tuple(new_uv) + tuple(new_uw)

        init = tuple(jnp.zeros((_L,), jnp.float32) for _ in range(2 * n_sl))
        sums = lax.fori_loop(0, D, body, init)
        for s in range(n_sl):
            sl = pl.ds(j * _CH + s * _L, _L)
            duv_v[sl] = 2.0 * t_uv[s] - sums[s]
            duw_v[sl] = 2.0 * t_uw[s] - sums[n_sl + s]

    pltpu.sync_copy(duv_v, duv_hbm.at[pl.ds(wid * bpw, bpw)])
    pltpu.sync_copy(duw_v, duw_hbm.at[pl.ds(wid * bpw, bpw)])


def _acosh_body(duv_ref, duw_ref, ouv_ref, ouw_ref):
    for s, o in ((duv_ref, ouv_ref), (duw_ref, ouw_ref)):
        d = jnp.maximum(s[...], 1.0 + 1e-07)
        o[...] = jnp.log(d + jnp.sqrt(d * d - 1.0))


def kernel(theta, parent, child, unrelated):
    B = parent.shape[0]
    V, D = theta.shape
    bpw = B // _NW
    n_chunks = bpw // _CH
    idx2 = (B // _CH, _CH)
    x = theta.swapaxes(0, 1)
    p2 = parent.astype(jnp.int32).reshape(idx2)
    c2 = child.astype(jnp.int32).reshape(idx2)
    u2 = unrelated.astype(jnp.int32).reshape(idx2)

    sc = pl.kernel(
        _sc_body,
        out_type=(jax.ShapeDtypeStruct((B,), jnp.float32),
                  jax.ShapeDtypeStruct((B,), jnp.float32)),
        mesh=plsc.VectorSubcoreMesh(core_axis_name="c", subcore_axis_name="s",
                                    num_cores=_NC, num_subcores=_NS),
        compiler_params=pltpu.CompilerParams(needs_layout_passes=False,
                                             use_tc_tiling_on_sc=False),
        scratch_types=[
            pltpu.VMEM((n_chunks, _CH), jnp.int32),
            pltpu.VMEM((n_chunks, _CH), jnp.int32),
            pltpu.VMEM((n_chunks, _CH), jnp.int32),
            pltpu.VMEM((D, _CH), jnp.float32),
            pltpu.VMEM((D, _CH), jnp.float32),
            pltpu.VMEM((D, _CH), jnp.float32),
            pltpu.VMEM((D, _CH), jnp.float32),
            pltpu.VMEM((D, _CH), jnp.float32),
            pltpu.VMEM((D, _CH), jnp.float32),
            pltpu.VMEM((bpw,), jnp.float32),
            pltpu.VMEM((bpw,), jnp.float32),
            pltpu.SemaphoreType.DMA,
        ],
    )
    duv, duw = sc(x, p2, c2, u2)

    tc = pl.pallas_call(
        _acosh_body,
        out_shape=(jax.ShapeDtypeStruct((B // _CH, _CH), jnp.float32),
                   jax.ShapeDtypeStruct((B // _CH, _CH), jnp.float32)),
    )
    ouv, ouw = tc(duv.reshape(B // _CH, _CH), duw.reshape(B // _CH, _CH))
    return ouv.reshape(B), ouw.reshape(B)


# speedup vs baseline: 4.8600x; 4.8600x over previous
import jax
import jax.numpy as jnp
from jax import lax
from jax.experimental import pallas as pl
from jax.experimental.pallas import tpu as pltpu
from jax.experimental.pallas import tpu_sc as plsc

_NC = 2
_NS = 16
_NW = _NC * _NS
_L = 16
_CH = 128


def _sc_body(theta4_hbm, pidx_hbm, cidx_hbm, uidx_hbm, duv_hbm, duw_hbm,
             pidx_v, cidx_v, uidx_v, pblk_v, cblk_v, ublk_v,
             ub0, ub1, vb0, vb1, wb0, wb1, duv_v, duw_v, sem):
    D = 32
    n_chunks = pidx_v.shape[0]
    bpw = n_chunks * _CH
    wid = lax.axis_index("s") * _NC + lax.axis_index("c")
    base_chunk = wid * n_chunks

    pltpu.sync_copy(pidx_hbm.at[pl.ds(base_chunk, n_chunks)], pidx_v)
    pltpu.sync_copy(cidx_hbm.at[pl.ds(base_chunk, n_chunks)], cidx_v)
    pltpu.sync_copy(uidx_hbm.at[pl.ds(base_chunk, n_chunks)], uidx_v)

    for src, dst in ((pidx_v, pblk_v), (cidx_v, cblk_v), (uidx_v, ublk_v)):
        for j in range(n_chunks):
            for s in range(_CH // _L):
                sl = pl.ds(s * _L, _L)
                dst.at[j][sl] = lax.shift_right_logical(src.at[j][sl], 2)

    bufs = ((ub0, vb0, wb0), (ub1, vb1, wb1))

    def fire(j):
        u_b, v_b, w_b = bufs[j % 2]
        return (
            pltpu.async_copy(theta4_hbm.at[pblk_v.at[j]], u_b, sem),
            pltpu.async_copy(theta4_hbm.at[cblk_v.at[j]], v_b, sem),
            pltpu.async_copy(theta4_hbm.at[ublk_v.at[j]], w_b, sem),
        )

    lane = lax.iota(jnp.int32, _L)
    cur = fire(0)
    for j in range(n_chunks):
        for dsc in cur:
            dsc.wait()
        if j + 1 < n_chunks:
            cur = fire(j + 1)
        u_b, v_b, w_b = bufs[j % 2]
        for s in range(_CH // _L):
            sl = pl.ds(s * _L, _L)
            row = jnp.full((_L,), s * _L, jnp.int32) + lane
            offu = lax.shift_left(pidx_v.at[j][sl] & 3, 5)
            offv = lax.shift_left(cidx_v.at[j][sl] & 3, 5)
            offw = lax.shift_left(uidx_v.at[j][sl] & 3, 5)
            u0 = plsc.load_gather(u_b, [row, offu])
            v0 = plsc.load_gather(v_b, [row, offv])
            w0 = plsc.load_gather(w_b, [row, offw])

            def body(dcol, accs, row=row, offu=offu, offv=offv, offw=offw,
                     u_b=u_b, v_b=v_b, w_b=w_b):
                a_uv, a_uw = accs
                ud = plsc.load_gather(u_b, [row, offu + dcol])
                vd = plsc.load_gather(v_b, [row, offv + dcol])
                wd = plsc.load_gather(w_b, [row, offw + dcol])
                return (a_uv - ud * vd, a_uw - ud * wd)

            acc_uv, acc_uw = lax.fori_loop(1, D, body, (u0 * v0, u0 * w0))
            g = j * (_CH // _L) + s
            duv_v[pl.ds(g * _L, _L)] = acc_uv
            duw_v[pl.ds(g * _L, _L)] = acc_uw

    pltpu.sync_copy(duv_v, duv_hbm.at[pl.ds(wid * bpw, bpw)])
    pltpu.sync_copy(duw_v, duw_hbm.at[pl.ds(wid * bpw, bpw)])


def _acosh_body(duv_ref, duw_ref, ouv_ref, ouw_ref):
    for s, o in ((duv_ref, ouv_ref), (duw_ref, ouw_ref)):
        d = jnp.maximum(s[...], 1.0 + 1e-07)
        o[...] = jnp.log(d + jnp.sqrt(d * d - 1.0))


def kernel(theta, parent, child, unrelated):
    B = parent.shape[0]
    V, D = theta.shape
    bpw = B // _NW
    n_chunks = bpw // _CH
    idx2 = (B // _CH, _CH)
    theta4 = theta.reshape(V // 4, 4 * D)
    p2 = parent.astype(jnp.int32).reshape(idx2)
    c2 = child.astype(jnp.int32).reshape(idx2)
    u2 = unrelated.astype(jnp.int32).reshape(idx2)

    sc = pl.kernel(
        _sc_body,
        out_type=(jax.ShapeDtypeStruct((B,), jnp.float32),
                  jax.ShapeDtypeStruct((B,), jnp.float32)),
        mesh=plsc.VectorSubcoreMesh(core_axis_name="c", subcore_axis_name="s",
                                    num_cores=_NC, num_subcores=_NS),
        compiler_params=pltpu.CompilerParams(needs_layout_passes=False,
                                             use_tc_tiling_on_sc=True),
        scratch_types=[
            pltpu.VMEM((n_chunks, _CH), jnp.int32),
            pltpu.VMEM((n_chunks, _CH), jnp.int32),
            pltpu.VMEM((n_chunks, _CH), jnp.int32),
            pltpu.VMEM((n_chunks, _CH), jnp.int32),
            pltpu.VMEM((n_chunks, _CH), jnp.int32),
            pltpu.VMEM((n_chunks, _CH), jnp.int32),
            pltpu.VMEM((_CH, 4 * D), jnp.float32),
            pltpu.VMEM((_CH, 4 * D), jnp.float32),
            pltpu.VMEM((_CH, 4 * D), jnp.float32),
            pltpu.VMEM((_CH, 4 * D), jnp.float32),
            pltpu.VMEM((_CH, 4 * D), jnp.float32),
            pltpu.VMEM((_CH, 4 * D), jnp.float32),
            pltpu.VMEM((bpw,), jnp.float32),
            pltpu.VMEM((bpw,), jnp.float32),
            pltpu.SemaphoreType.DMA,
        ],
    )
    duv, duw = sc(theta4, p2, c2, u2)

    tc = pl.pallas_call(
        _acosh_body,
        out_shape=(jax.ShapeDtypeStruct((B // _CH, _CH), jnp.float32),
                   jax.ShapeDtypeStruct((B // _CH, _CH), jnp.float32)),
    )
    ouv, ouw = tc(duv.reshape(B // _CH, _CH), duw.reshape(B // _CH, _CH))
    return ouv.reshape(B), ouw.reshape(B)
